# baseline (speedup 1.0000x reference)
import functools

import jax
import jax.numpy as jnp
from jax import lax
from jax.experimental import pallas as pl
from jax.experimental.pallas import tpu as pltpu
from jax.experimental.pallas import tpu_sc as plsc

N = 4096
D = 256
K = 2048
NW = N // 2
NSLAB = NW // 128

_PK_ROWS = 256
_PK_CHUNK = 256


def _binpack_body(g_ref, b32_ref):
    for wc in range(NW // _PK_CHUNK):
        lo = (g_ref[:, pl.ds(wc * _PK_CHUNK, _PK_CHUNK)] != 0.0)
        hi = (g_ref[:, pl.ds(NW + wc * _PK_CHUNK, _PK_CHUNK)] != 0.0)
        w = (lo.astype(jnp.float32)
             + 65536.0 * hi.astype(jnp.float32)).astype(jnp.int32)
        for s in range(_PK_CHUNK // 128):
            b32_ref[:, wc * (_PK_CHUNK // 128) + s, :] = (
                w[:, s * 128:(s + 1) * 128])


def _binpack(g):
    return pl.pallas_call(
        _binpack_body,
        grid=(N // _PK_ROWS,),
        in_specs=[pl.BlockSpec((_PK_ROWS, N), lambda i: (i, 0))],
        out_specs=pl.BlockSpec((_PK_ROWS, NSLAB, 128), lambda i: (i, 0, 0)),
        out_shape=jax.ShapeDtypeStruct((N, NSLAB, 128), jnp.int32),
    )(g)


_TP_ROWS = 256


def _transpack_body(g_ref, bt32_ref):
    for wc in range(NW // _PK_CHUNK):
        tlo = (g_ref[pl.ds(wc * _PK_CHUNK, _PK_CHUNK), :] != 0.0)
        thi = (g_ref[pl.ds(NW + wc * _PK_CHUNK, _PK_CHUNK), :] != 0.0)
        w = (tlo.astype(jnp.float32).T
             + 65536.0 * thi.astype(jnp.float32).T).astype(jnp.int32)
        for s in range(_PK_CHUNK // 128):
            bt32_ref[:, wc * (_PK_CHUNK // 128) + s, :] = (
                w[:, s * 128:(s + 1) * 128])


def _transpack(g):
    return pl.pallas_call(
        _transpack_body,
        grid=(N // _TP_ROWS,),
        in_specs=[pl.BlockSpec((N, _TP_ROWS), lambda i: (0, i))],
        out_specs=pl.BlockSpec((_TP_ROWS, NSLAB, 128), lambda i: (i, 0, 0)),
        out_shape=jax.ShapeDtypeStruct((N, NSLAB, 128), jnp.int32),
    )(g)


_RANK_ROWS = 128
_RANK_CHUNK = 512


def _rank_body(scol_ref, srow_ref, rank_ref):
    pid = pl.program_id(0)
    scol = scol_ref[...]
    gidx = (jax.lax.broadcasted_iota(jnp.int32, (_RANK_ROWS, 1), 0)
            + pid * _RANK_ROWS)

    def body(kc, acc):
        srow = srow_ref[:, pl.ds(kc * _RANK_CHUNK, _RANK_CHUNK)]
        jidx = (jax.lax.broadcasted_iota(jnp.int32, (1, _RANK_CHUNK), 1)
                + kc * _RANK_CHUNK)
        beat = (srow > scol) | ((srow == scol) & (jidx < gidx))
        return acc + jnp.sum(beat.astype(jnp.float32), axis=1, keepdims=True)

    cnt = jax.lax.fori_loop(0, N // _RANK_CHUNK, body,
                            jnp.zeros((_RANK_ROWS, 1), jnp.float32))
    rank_ref[...] = cnt.astype(jnp.int32)


def _ranks(scores_col, scores_row):
    return pl.pallas_call(
        _rank_body,
        grid=(N // _RANK_ROWS,),
        in_specs=[
            pl.BlockSpec((_RANK_ROWS, 1), lambda i: (i, 0)),
            pl.BlockSpec((1, N), lambda i: (0, 0)),
        ],
        out_specs=pl.BlockSpec((_RANK_ROWS, 1), lambda i: (i, 0)),
        out_shape=jax.ShapeDtypeStruct((N, 1), jnp.int32),
    )(scores_col, scores_row)


_INV_ROWS = 128
_INV_CHUNK = 512


def _invert_body(rank_ref, srow_ref, idx_ref, val_ref):
    pid = pl.program_id(0)
    pcol = (jax.lax.broadcasted_iota(jnp.int32, (_INV_ROWS, 1), 0)
            + pid * _INV_ROWS)

    def body(kc, carry):
        idx_acc, val_acc = carry
        rr = rank_ref[:, pl.ds(kc * _INV_CHUNK, _INV_CHUNK)]
        ss = srow_ref[:, pl.ds(kc * _INV_CHUNK, _INV_CHUNK)]
        jidx = (jax.lax.broadcasted_iota(jnp.int32, (1, _INV_CHUNK), 1)
                + kc * _INV_CHUNK)
        hit = rr == pcol
        idx_acc = idx_acc + jnp.sum(
            jnp.where(hit, jidx, 0), axis=1, keepdims=True)
        val_acc = val_acc + jnp.sum(
            jnp.where(hit, ss, 0.0), axis=1, keepdims=True)
        return idx_acc, val_acc

    idx_acc, val_acc = jax.lax.fori_loop(
        0, N // _INV_CHUNK, body,
        (jnp.zeros((_INV_ROWS, 1), jnp.int32),
         jnp.zeros((_INV_ROWS, 1), jnp.float32)))
    idx_ref[...] = idx_acc
    val_ref[...] = val_acc


def _invert(rank_row, scores_row):
    return pl.pallas_call(
        _invert_body,
        grid=(K // _INV_ROWS,),
        in_specs=[
            pl.BlockSpec((1, N), lambda i: (0, 0)),
            pl.BlockSpec((1, N), lambda i: (0, 0)),
        ],
        out_specs=[
            pl.BlockSpec((_INV_ROWS, 1), lambda i: (i, 0)),
            pl.BlockSpec((_INV_ROWS, 1), lambda i: (i, 0)),
        ],
        out_shape=[
            jax.ShapeDtypeStruct((K, 1), jnp.int32),
            jax.ShapeDtypeStruct((K, 1), jnp.float32),
        ],
    )(rank_row, scores_row)


_SC_ROWS_PER_W = K // 32
_SC_CHUNK = 32


def _sc_gather(idx, b32, bt32):
    mesh = plsc.VectorSubcoreMesh(core_axis_name="c", subcore_axis_name="s")

    @functools.partial(
        pl.kernel,
        out_type=[
            jax.ShapeDtypeStruct((K, NSLAB, 128), jnp.int32),
            jax.ShapeDtypeStruct((K, NSLAB, 128), jnp.int32),
        ],
        mesh=mesh,
        scratch_types=[
            pltpu.VMEM((_SC_CHUNK,), jnp.int32),
            pltpu.VMEM((_SC_CHUNK, NSLAB, 128), jnp.int32),
            pltpu.SemaphoreType.DMA,
        ],
    )
    def gather(idx_hbm, b_hbm, bt_hbm, a_hbm, r_hbm, idx_v, rows_v, sem):
        wid = lax.axis_index("s") * 2 + lax.axis_index("c")
        base0 = wid * _SC_ROWS_PER_W
        for c in range(_SC_ROWS_PER_W // _SC_CHUNK):
            base = base0 + c * _SC_CHUNK
            pltpu.sync_copy(idx_hbm.at[pl.ds(base, _SC_CHUNK)], idx_v)
            pltpu.async_copy(b_hbm.at[idx_v], rows_v, sem).wait()
            pltpu.sync_copy(rows_v, a_hbm.at[pl.ds(base, _SC_CHUNK)])
            pltpu.async_copy(bt_hbm.at[idx_v], rows_v, sem).wait()
            pltpu.sync_copy(rows_v, r_hbm.at[pl.ds(base, _SC_CHUNK)])

    return gather(idx, b32, bt32)


_NH_BP = 512
_NH_BK = 512


def _newh_body(rank_ref, h_ref, val_ref, out_ref):
    p, kc = pl.program_id(0), pl.program_id(1)
    pcol = (jax.lax.broadcasted_iota(jnp.int32, (_NH_BP, 1), 0) + p * _NH_BP)
    rr = rank_ref[...]
    oh = (rr == pcol).astype(jnp.float32)
    acc = jax.lax.dot_general(
        oh, h_ref[...], dimension_numbers=(((1,), (0,)), ((), ())),
        precision=jax.lax.Precision.HIGHEST,
        preferred_element_type=jnp.float32)

    @pl.when(kc == 0)
    def _():
        out_ref[...] = jnp.zeros_like(out_ref)

    @pl.when(kc < pl.num_programs(1) - 1)
    def _():
        out_ref[...] += acc

    @pl.when(kc == pl.num_programs(1) - 1)
    def _():
        out_ref[...] = (out_ref[...] + acc) * val_ref[...]


def _newh(rank_row, h, val2d):
    return pl.pallas_call(
        _newh_body,
        grid=(K // _NH_BP, N // _NH_BK),
        in_specs=[
            pl.BlockSpec((1, _NH_BK), lambda p, kc: (0, kc)),
            pl.BlockSpec((_NH_BK, D), lambda p, kc: (kc, 0)),
            pl.BlockSpec((_NH_BP, 1), lambda p, kc: (p, 0)),
        ],
        out_specs=pl.BlockSpec((_NH_BP, D), lambda p, kc: (p, 0)),
        out_shape=jax.ShapeDtypeStruct((K, D), jnp.float32),
    )(rank_row, h, val2d)


_MM_BP = 512
_MM_BQ = 512
_MM_SLABS = 8


def _mm_unpack(w):
    lo = (w & 65535).astype(jnp.bfloat16)
    hi = (w >> 16).astype(jnp.bfloat16)
    return jnp.concatenate([lo, hi], axis=1)


def _matmul_body(a_ref, r_ref, cnt_ref):
    ki = pl.program_id(2)

    @pl.when(ki == 0)
    def _():
        cnt_ref[...] = jnp.zeros_like(cnt_ref)

    for s in range(_MM_SLABS):
        ab = _mm_unpack(a_ref[:, s, :])
        rb = _mm_unpack(r_ref[:, s, :])
        acc = jax.lax.dot_general(
            ab, rb, dimension_numbers=(((1,), (1,)), ((), ())),
            preferred_element_type=jnp.float32)
        cnt_ref[...] += acc.astype(jnp.bfloat16)


def _matmul(a32, r32):
    return pl.pallas_call(
        _matmul_body,
        grid=(K // _MM_BP, K // _MM_BQ, NSLAB // _MM_SLABS),
        in_specs=[
            pl.BlockSpec((_MM_BP, _MM_SLABS, 128), lambda p, q, k: (p, k, 0)),
            pl.BlockSpec((_MM_BQ, _MM_SLABS, 128), lambda p, q, k: (q, k, 0)),
        ],
        out_specs=pl.BlockSpec((_MM_BP, _MM_BQ), lambda p, q, k: (p, q)),
        out_shape=jax.ShapeDtypeStruct((K, K), jnp.bfloat16),
    )(a32, r32)


_NORM_ROWS = 256
_NORM_CHUNK = 512


def _norm_body(cnt_ref, out_ref):
    nchunks = K // _NORM_CHUNK

    def degbody(kc, acc):
        c = cnt_ref[:, pl.ds(kc * _NORM_CHUNK, _NORM_CHUNK)]
        un = (c > 0).astype(jnp.float32)
        return acc + jnp.sum(un, axis=1, keepdims=True)

    deg = jax.lax.fori_loop(0, nchunks, degbody,
                            jnp.zeros((_NORM_ROWS, 1), jnp.float32))
    recip = 1.0 / deg

    def outbody(kc, _):
        c = cnt_ref[:, pl.ds(kc * _NORM_CHUNK, _NORM_CHUNK)]
        un = (c > 0).astype(jnp.float32)
        out_ref[:, pl.ds(kc * _NORM_CHUNK, _NORM_CHUNK)] = un * recip
        return 0

    jax.lax.fori_loop(0, nchunks, outbody, 0)


def _normalize(cnt):
    return pl.pallas_call(
        _norm_body,
        grid=(K // _NORM_ROWS,),
        in_specs=[pl.BlockSpec((_NORM_ROWS, K), lambda i: (i, 0))],
        out_specs=pl.BlockSpec((_NORM_ROWS, K), lambda i: (i, 0)),
        out_shape=jax.ShapeDtypeStruct((K, K), jnp.float32),
    )(cnt)


@jax.jit
def kernel(g, h, W, b):
    weights = jnp.squeeze(h @ W + b, axis=-1)
    scores = jax.nn.sigmoid(weights)

    scores_row = scores.reshape(1, N)
    scores_col = scores.reshape(N, 1)

    b32 = _binpack(g)
    bt32 = _transpack(g)
    rank = _ranks(scores_col, scores_row)
    rank_row = rank.reshape(1, N)
    idx2d, val2d = _invert(rank_row, scores_row)
    idx = idx2d.reshape(K)
    a32, r32 = _sc_gather(idx, b32, bt32)
    new_h = _newh(rank_row, h, val2d)
    cnt = _matmul(a32, r32)
    g_out = _normalize(cnt)
    return (g_out, new_h, idx)
